# baseline (speedup 1.0000x reference)
import jax
import jax.numpy as jnp
from jax import lax
from jax.experimental import pallas as pl
from jax.experimental.pallas import tpu as pltpu
from jax.experimental.pallas import tpu_sc as plsc

_C, _H, _F, _K = 4096, 3, 3, 16
_D = _H + _F
_NC, _NS = 2, 16
_NW = _NC * _NS
_SPW = _C // _NW
_L = 16
_G = _SPW // _L


def _to_idx(v):
    f = v.astype(jnp.int32).astype(jnp.float32)
    f = jnp.minimum(jnp.maximum(f, 0.0), 0.2)
    return f.astype(jnp.int32)


def _sc_count(xi_hbm, xo_hbm, bi_hbm, bo_hbm, out_hbm,
              xi_v, xo_v, bi_v, bo_v, res_v):
    wid = lax.axis_index("s") * _NC + lax.axis_index("c")
    base = wid * _SPW
    pltpu.sync_copy(xi_hbm.at[pl.ds(base, _SPW)], xi_v)
    pltpu.sync_copy(xo_hbm.at[pl.ds(base, _SPW)], xo_v)
    pltpu.sync_copy(bi_hbm.at[pl.ds(base, _SPW)], bi_v)
    pltpu.sync_copy(bo_hbm.at[pl.ds(base, _SPW)], bo_v)

    def sample(s, acc):
        lin = jnp.zeros((_L,), jnp.int32)
        for d in range(_H):
            v = xi_v[s, pl.ds(d * _K, _K)] + bi_v[s, pl.ds(d * _K, _K)]
            lin = lin + _to_idx(v) * (_K ** (_D - 1 - d))
        for d in range(_F):
            v = xo_v[s, pl.ds(d * _K, _K)] + bo_v[s, pl.ds(d * _K, _K)]
            lin = lin + _to_idx(v) * (_K ** (_F - 1 - d))
        ind = 1 - jnp.minimum(lin, 1)
        res_v[pl.ds(s * _L, _L)] = ind.astype(jnp.float32)
        return acc

    lax.fori_loop(0, _SPW, sample, jnp.zeros((_L,), jnp.float32))
    pltpu.sync_copy(res_v, out_hbm.at[pl.ds(base * _L, _SPW * _L)])


def _fill_body(out_ref):
    out_ref[...] = jnp.zeros_like(out_ref)


def _insert_body(filled_ref, ind_ref, out_ref):
    del filled_ref
    ind = ind_ref[...]
    gl = jax.lax.broadcasted_iota(jnp.int32, (128, 128 // _K), 0) // _K
    gc = jax.lax.broadcasted_iota(jnp.int32, (128, 128 // _K), 1)
    grp = (gl == gc).astype(jnp.float32)
    per_sample = jnp.dot(ind, grp)
    count = jnp.sum(jnp.minimum(per_sample, 1.0))
    r = jax.lax.broadcasted_iota(jnp.int32, (_K, _K), 0)
    c = jax.lax.broadcasted_iota(jnp.int32, (_K, _K), 1)
    tile = jnp.where((r == 0) & (c == 0), count * (1.0 / _C), 0.0)
    out_ref[0, 0, 0, 0, :, :] = tile


def kernel(input_tensor, output_tensor, bias_input, bias_output):
    xi = input_tensor.reshape(_C, _H * _K)
    xo = output_tensor.reshape(_C, _F * _K)
    bi = bias_input.reshape(_C, _H * _K)
    bo = bias_output.reshape(_C, _F * _K)

    sc_fn = pl.kernel(
        _sc_count,
        out_type=jax.ShapeDtypeStruct((_C * _K,), jnp.float32),
        mesh=plsc.VectorSubcoreMesh(core_axis_name="c", subcore_axis_name="s"),
        scratch_types=[
            pltpu.VMEM((_SPW, _H * _K), jnp.float32),
            pltpu.VMEM((_SPW, _F * _K), jnp.float32),
            pltpu.VMEM((_SPW, _H * _K), jnp.float32),
            pltpu.VMEM((_SPW, _F * _K), jnp.float32),
            pltpu.VMEM((_SPW * _L,), jnp.float32),
        ],
    )
    indicators = sc_fn(xi, xo, bi, bo).reshape(_C * _K // 128, 128)

    filled = pl.pallas_call(
        _fill_body,
        grid=(_K, 2),
        out_specs=pl.BlockSpec(
            (1, 8, _K, _K, _K, _K), lambda i, j: (i, j, 0, 0, 0, 0)
        ),
        out_shape=jax.ShapeDtypeStruct((_K,) * _D, jnp.float32),
        compiler_params=pltpu.CompilerParams(
            dimension_semantics=("parallel", "parallel"),
        ),
    )()

    out = pl.pallas_call(
        _insert_body,
        grid=(1,),
        in_specs=[
            pl.BlockSpec((1, 1, 1, 1, _K, _K), lambda i: (0,) * _D),
            pl.BlockSpec((_C * _K // 128, 128), lambda i: (0, 0)),
        ],
        out_specs=pl.BlockSpec((1, 1, 1, 1, _K, _K), lambda i: (0,) * _D),
        out_shape=jax.ShapeDtypeStruct((_K,) * _D, jnp.float32),
        input_output_aliases={0: 0},
    )(filled, indicators)
    return out


# speedup vs baseline: 1.0156x; 1.0156x over previous
import jax
import jax.numpy as jnp
from jax import lax
from jax.experimental import pallas as pl
from jax.experimental.pallas import tpu as pltpu
from jax.experimental.pallas import tpu_sc as plsc

_C, _H, _F, _K = 4096, 3, 3, 16
_D = _H + _F
_NC, _NS = 2, 16
_NW = _NC * _NS
_SPW = _C // _NW
_L = 16
_G = _SPW // _L


def _to_idx(v):
    f = v.astype(jnp.int32).astype(jnp.float32)
    f = jnp.minimum(jnp.maximum(f, 0.0), 0.2)
    return f.astype(jnp.int32)


def _sc_count(xi_hbm, xo_hbm, bi_hbm, bo_hbm, out_hbm,
              xi_v, xo_v, bi_v, bo_v, res_v):
    wid = lax.axis_index("s") * _NC + lax.axis_index("c")
    pltpu.sync_copy(xi_hbm.at[wid], xi_v)
    pltpu.sync_copy(xo_hbm.at[wid], xo_v)
    pltpu.sync_copy(bi_hbm.at[wid], bi_v)
    pltpu.sync_copy(bo_hbm.at[wid], bo_v)

    def group(j, cnt):
        hit = jnp.zeros((_L,), jnp.int32)
        for k in range(_K):
            lin = jnp.zeros((_L,), jnp.int32)
            for d in range(_H):
                v = (xi_v[d * _K + k, pl.ds(j * _L, _L)]
                     + bi_v[d * _K + k, pl.ds(j * _L, _L)])
                lin = lin + _to_idx(v) * (_K ** (_D - 1 - d))
            for d in range(_F):
                v = (xo_v[d * _K + k, pl.ds(j * _L, _L)]
                     + bo_v[d * _K + k, pl.ds(j * _L, _L)])
                lin = lin + _to_idx(v) * (_K ** (_F - 1 - d))
            hit = jnp.maximum(hit, 1 - jnp.minimum(lin, 1))
        return cnt + hit.astype(jnp.float32)

    cnt = lax.fori_loop(0, _G, group, jnp.zeros((_L,), jnp.float32))
    res_v[...] = cnt
    pltpu.sync_copy(res_v, out_hbm.at[wid])


def _fill_body(out_ref):
    out_ref[...] = jnp.zeros_like(out_ref)


def _insert_body(filled_ref, p_ref, out_ref):
    del filled_ref
    count = jnp.sum(p_ref[...])
    r = jax.lax.broadcasted_iota(jnp.int32, (_K, _K), 0)
    c = jax.lax.broadcasted_iota(jnp.int32, (_K, _K), 1)
    tile = jnp.where((r == 0) & (c == 0), count * (1.0 / _C), 0.0)
    out_ref[0, 0, 0, 0, :, :] = tile


def _sample_minor(x):
    return x.reshape(_NW, _SPW, -1).transpose(0, 2, 1)


def kernel(input_tensor, output_tensor, bias_input, bias_output):
    xi = _sample_minor(input_tensor.reshape(_C, _H * _K))
    xo = _sample_minor(output_tensor.reshape(_C, _F * _K))
    bi = _sample_minor(bias_input.reshape(_C, _H * _K))
    bo = _sample_minor(bias_output.reshape(_C, _F * _K))

    sc_fn = pl.kernel(
        _sc_count,
        out_type=jax.ShapeDtypeStruct((_NW, _L), jnp.float32),
        mesh=plsc.VectorSubcoreMesh(core_axis_name="c", subcore_axis_name="s"),
        scratch_types=[
            pltpu.VMEM((_H * _K, _SPW), jnp.float32),
            pltpu.VMEM((_F * _K, _SPW), jnp.float32),
            pltpu.VMEM((_H * _K, _SPW), jnp.float32),
            pltpu.VMEM((_F * _K, _SPW), jnp.float32),
            pltpu.VMEM((_L,), jnp.float32),
        ],
    )
    partials = sc_fn(xi, xo, bi, bo)

    filled = pl.pallas_call(
        _fill_body,
        grid=(_K, 4),
        out_specs=pl.BlockSpec(
            (1, 4, _K, _K, _K, _K), lambda i, j: (i, j, 0, 0, 0, 0)
        ),
        out_shape=jax.ShapeDtypeStruct((_K,) * _D, jnp.float32),
        compiler_params=pltpu.CompilerParams(
            dimension_semantics=("parallel", "parallel"),
        ),
    )()

    out = pl.pallas_call(
        _insert_body,
        grid=(1,),
        in_specs=[
            pl.BlockSpec((1, 1, 1, 1, _K, _K), lambda i: (0,) * _D),
            pl.BlockSpec((_NW, _L), lambda i: (0, 0)),
        ],
        out_specs=pl.BlockSpec((1, 1, 1, 1, _K, _K), lambda i: (0,) * _D),
        out_shape=jax.ShapeDtypeStruct((_K,) * _D, jnp.float32),
        input_output_aliases={0: 0},
    )(filled, partials)
    return out
